# baseline (speedup 1.0000x reference)
import jax
import jax.numpy as jnp
from jax import lax
from jax.experimental import pallas as pl
from jax.experimental.pallas import tpu as pltpu
from jax.experimental.pallas import tpu_sc as plsc

_B = 1024
_HID = 32
_LAMBDA = 1e-05
_NU = 1000000
_NI = 100000

_NC = 2
_NS = 16
_NW = _NC * _NS
_BPW = _B // _NW

_RING = 16

_BN = 2048


def _make_gather_body(n_rows):
    cmax = n_rows // 128 - 1
    toff = (cmax + 1) * 128
    tw = n_rows - toff

    def body(et_hbm, idx_hbm, bias_hbm,
             e_out, b_out,
             idx_v, e_loc, tail, b_v, bsem, bufs, sems, tsem):
        wid = lax.axis_index("s") * _NC + lax.axis_index("c")
        base = wid * _BPW
        pltpu.sync_copy(idx_hbm.at[pl.ds(base, _BPW)], idx_v)
        cb = pltpu.async_copy(bias_hbm.at[idx_v], b_v, bsem)
        ctail = pltpu.async_copy(et_hbm.at[:, pl.ds(toff, tw)], tail, tsem)

        iota16 = lax.iota(jnp.int32, 16)

        def issue(j):
            chunk = idx_v[pl.ds((j // 16) * 16, 16)]
            xj = chunk[j % 16]
            c = jnp.minimum(xj // 128, cmax)
            off = pl.multiple_of(c * 128, 128)
            slot = j % _RING
            return pltpu.async_copy(et_hbm.at[:, pl.ds(off, 128)],
                                    bufs[slot], sems[slot])

        def extract(j):
            chunk = idx_v[pl.ds((j // 16) * 16, 16)]
            xj = chunk[j % 16]
            c = jnp.minimum(xj // 128, cmax)
            lane_m = jnp.minimum(xj - c * 128, 127)
            lane_t = jnp.clip(xj - toff, 0, tw - 1)
            in_tail = jnp.full((16,), xj >= toff, jnp.bool_)
            slot = j % _RING
            for h in range(2):
                rows = iota16 + h * 16
                vm = plsc.load_gather(
                    bufs[slot], [rows, jnp.full((16,), lane_m, jnp.int32)])
                vt = plsc.load_gather(
                    tail, [rows, jnp.full((16,), lane_t, jnp.int32)])
                e_loc[j, pl.ds(h * 16, 16)] = jnp.where(in_tail, vt, vm)

        pend = [None] * _RING
        for j in range(_RING):
            pend[j] = issue(j)
        ctail.wait()
        for j in range(_BPW):
            pend[j % _RING].wait()
            extract(j)
            nj = j + _RING
            if nj < _BPW:
                pend[nj % _RING] = issue(nj)

        pltpu.sync_copy(e_loc, e_out.at[pl.ds(base, _BPW), :])
        cb.wait()
        pltpu.sync_copy(b_v, b_out.at[pl.ds(base, _BPW)])

    return body, tw


def _sc_gather(et, idx, bias, n_rows):
    body, tw = _make_gather_body(n_rows)
    mesh = plsc.VectorSubcoreMesh(
        core_axis_name="c", subcore_axis_name="s",
        num_cores=_NC, num_subcores=_NS)
    f = pl.kernel(
        body,
        out_type=(
            jax.ShapeDtypeStruct((_B, _HID), jnp.float32),
            jax.ShapeDtypeStruct((_B,), jnp.float32),
        ),
        mesh=mesh,
        scratch_types=[
            pltpu.VMEM((_BPW,), jnp.int32),
            pltpu.VMEM((_BPW, _HID), jnp.float32),
            pltpu.VMEM((_HID, tw), jnp.float32),
            pltpu.VMEM((_BPW,), jnp.float32),
            pltpu.SemaphoreType.DMA,
            [pltpu.VMEM((_HID, 128), jnp.float32)] * _RING,
            [pltpu.SemaphoreType.DMA] * _RING,
            pltpu.SemaphoreType.DMA,
        ],
        compiler_params=pltpu.CompilerParams(use_tc_tiling_on_sc=True,
                                             needs_layout_passes=False),
    )
    return f(et, idx, bias)


def _score_body(ue_ref, eti_ref, ibias_ref, score_ref, at_ref):
    pid = pl.program_id(0)

    @pl.when(pid == 0)
    def _prep():
        at_ref[...] = ue_ref[...].T

    sc = lax.dot_general(eti_ref[...], at_ref[...],
                         (((0,), (0,)), ((), ())),
                         preferred_element_type=jnp.float32)
    score_ref[...] = sc + ibias_ref[...][:, None]


def _tc_score(ue, eti, item_bias):
    grid = (pl.cdiv(_NI, _BN),)
    return pl.pallas_call(
        _score_body,
        grid=grid,
        in_specs=[
            pl.BlockSpec((_B, _HID), lambda n: (0, 0)),
            pl.BlockSpec((_HID, _BN), lambda n: (0, n)),
            pl.BlockSpec((_BN,), lambda n: (n,)),
        ],
        out_specs=pl.BlockSpec((_BN, _B), lambda n: (n, 0)),
        out_shape=jax.ShapeDtypeStruct((_NI, _B), jnp.float32),
        scratch_shapes=[pltpu.VMEM((_HID, _B), jnp.float32)],
        compiler_params=pltpu.CompilerParams(
            dimension_semantics=("parallel",),
        ),
    )(ue, eti, item_bias)


def _small_body(ue_ref, ie_ref, ub_ref, ib_ref, y_ref, gb_ref,
                s_ref, loss_ref, diff_ref):
    ue = ue_ref[...]
    ie = ie_ref[...]
    ub = ub_ref[...]
    ib = ib_ref[...]
    s = jnp.sum(ue * ie, axis=1) + ub + ib + gb_ref[0, 0]
    d = s - y_ref[...]
    s_ref[...] = s
    diff_ref[...] = d
    l2 = (jnp.mean(ue * ue) + jnp.mean(ie * ie)
          + jnp.mean(ub * ub) + jnp.mean(ib * ib))
    loss_ref[0, 0] = jnp.mean(d * d) + _LAMBDA * l2


def _tc_small(ue, ie, ub, ib, y, gb2d):
    return pl.pallas_call(
        _small_body,
        in_specs=[
            pl.BlockSpec((_B, _HID), lambda: (0, 0)),
            pl.BlockSpec((_B, _HID), lambda: (0, 0)),
            pl.BlockSpec((_B,), lambda: (0,)),
            pl.BlockSpec((_B,), lambda: (0,)),
            pl.BlockSpec((_B,), lambda: (0,)),
            pl.BlockSpec((1, 1), lambda: (0, 0), memory_space=pltpu.SMEM),
        ],
        out_specs=[
            pl.BlockSpec((_B,), lambda: (0,)),
            pl.BlockSpec((1, 1), lambda: (0, 0), memory_space=pltpu.SMEM),
            pl.BlockSpec((_B,), lambda: (0,)),
        ],
        out_shape=[
            jax.ShapeDtypeStruct((_B,), jnp.float32),
            jax.ShapeDtypeStruct((1, 1), jnp.float32),
            jax.ShapeDtypeStruct((_B,), jnp.float32),
        ],
    )(ue, ie, ub, ib, y, gb2d)


def kernel(u, i, y, user_emb, item_emb, user_bias, item_bias, global_bias):
    et = user_emb.T
    eti = item_emb.T
    ue, ub = _sc_gather(et, u, user_bias, _NU)
    ie, ib = _sc_gather(eti, i, item_bias, _NI)
    gb2d = jnp.reshape(global_bias, (1, 1)).astype(jnp.float32)
    score_t = _tc_score(ue, eti, item_bias)
    s, loss, diff = _tc_small(ue, ie, ub, ib, y, gb2d)
    return s, score_t.T, jnp.reshape(loss, ()), diff


# speedup vs baseline: 1.0034x; 1.0034x over previous
import jax
import jax.numpy as jnp
from jax import lax
from jax.experimental import pallas as pl
from jax.experimental.pallas import tpu as pltpu
from jax.experimental.pallas import tpu_sc as plsc

_B = 1024
_HID = 32
_LAMBDA = 1e-05
_NU = 1000000
_NI = 100000

_NC = 2
_NS = 16
_NW = _NC * _NS
_BPW = _B // _NW

_RING = 16

_BN = 4096


def _make_gather_body(n_rows):
    cmax = n_rows // 128 - 1
    toff = (cmax + 1) * 128
    tw = n_rows - toff

    def body(et_hbm, idx_hbm, bias_hbm,
             e_out, b_out,
             idx_v, e_loc, tail, b_v, bsem, bufs, sems, tsem):
        wid = lax.axis_index("s") * _NC + lax.axis_index("c")
        base = wid * _BPW
        pltpu.sync_copy(idx_hbm.at[pl.ds(base, _BPW)], idx_v)
        cb = pltpu.async_copy(bias_hbm.at[idx_v], b_v, bsem)
        ctail = pltpu.async_copy(et_hbm.at[:, pl.ds(toff, tw)], tail, tsem)

        iota16 = lax.iota(jnp.int32, 16)

        def issue(j):
            chunk = idx_v[pl.ds((j // 16) * 16, 16)]
            xj = chunk[j % 16]
            c = jnp.minimum(xj // 128, cmax)
            off = pl.multiple_of(c * 128, 128)
            slot = j % _RING
            return pltpu.async_copy(et_hbm.at[:, pl.ds(off, 128)],
                                    bufs[slot], sems[slot])

        def extract(j):
            chunk = idx_v[pl.ds((j // 16) * 16, 16)]
            xj = chunk[j % 16]
            c = jnp.minimum(xj // 128, cmax)
            lane_m = jnp.minimum(xj - c * 128, 127)
            lane_t = jnp.clip(xj - toff, 0, tw - 1)
            in_tail = jnp.full((16,), xj >= toff, jnp.bool_)
            slot = j % _RING
            for h in range(2):
                rows = iota16 + h * 16
                vm = plsc.load_gather(
                    bufs[slot], [rows, jnp.full((16,), lane_m, jnp.int32)])
                vt = plsc.load_gather(
                    tail, [rows, jnp.full((16,), lane_t, jnp.int32)])
                e_loc[j, pl.ds(h * 16, 16)] = jnp.where(in_tail, vt, vm)

        pend = [None] * _RING
        for j in range(_RING):
            pend[j] = issue(j)
        ctail.wait()
        for j in range(_BPW):
            pend[j % _RING].wait()
            extract(j)
            nj = j + _RING
            if nj < _BPW:
                pend[nj % _RING] = issue(nj)

        pltpu.sync_copy(e_loc, e_out.at[pl.ds(base, _BPW), :])
        cb.wait()
        pltpu.sync_copy(b_v, b_out.at[pl.ds(base, _BPW)])

    return body, tw


def _sc_gather(et, idx, bias, n_rows):
    body, tw = _make_gather_body(n_rows)
    mesh = plsc.VectorSubcoreMesh(
        core_axis_name="c", subcore_axis_name="s",
        num_cores=_NC, num_subcores=_NS)
    f = pl.kernel(
        body,
        out_type=(
            jax.ShapeDtypeStruct((_B, _HID), jnp.float32),
            jax.ShapeDtypeStruct((_B,), jnp.float32),
        ),
        mesh=mesh,
        scratch_types=[
            pltpu.VMEM((_BPW,), jnp.int32),
            pltpu.VMEM((_BPW, _HID), jnp.float32),
            pltpu.VMEM((_HID, tw), jnp.float32),
            pltpu.VMEM((_BPW,), jnp.float32),
            pltpu.SemaphoreType.DMA,
            [pltpu.VMEM((_HID, 128), jnp.float32)] * _RING,
            [pltpu.SemaphoreType.DMA] * _RING,
            pltpu.SemaphoreType.DMA,
        ],
        compiler_params=pltpu.CompilerParams(use_tc_tiling_on_sc=True,
                                             needs_layout_passes=False),
    )
    return f(et, idx, bias)


def _score_body(ue_ref, eti_ref, ibias_ref, score_ref, at_ref):
    pid = pl.program_id(0)

    @pl.when(pid == 0)
    def _prep():
        at_ref[...] = ue_ref[...].T

    sc = lax.dot_general(eti_ref[...], at_ref[...],
                         (((0,), (0,)), ((), ())),
                         preferred_element_type=jnp.float32)
    score_ref[...] = sc + ibias_ref[...][:, None]


def _tc_score(ue, eti, item_bias):
    grid = (pl.cdiv(_NI, _BN),)
    return pl.pallas_call(
        _score_body,
        grid=grid,
        in_specs=[
            pl.BlockSpec((_B, _HID), lambda n: (0, 0)),
            pl.BlockSpec((_HID, _BN), lambda n: (0, n)),
            pl.BlockSpec((_BN,), lambda n: (n,)),
        ],
        out_specs=pl.BlockSpec((_BN, _B), lambda n: (n, 0)),
        out_shape=jax.ShapeDtypeStruct((_NI, _B), jnp.float32),
        scratch_shapes=[pltpu.VMEM((_HID, _B), jnp.float32)],
        compiler_params=pltpu.CompilerParams(
            dimension_semantics=("parallel",),
        ),
    )(ue, eti, item_bias)


def _small_body(ue_ref, ie_ref, ub_ref, ib_ref, y_ref, gb_ref,
                s_ref, loss_ref, diff_ref):
    ue = ue_ref[...]
    ie = ie_ref[...]
    ub = ub_ref[...]
    ib = ib_ref[...]
    s = jnp.sum(ue * ie, axis=1) + ub + ib + gb_ref[0, 0]
    d = s - y_ref[...]
    s_ref[...] = s
    diff_ref[...] = d
    l2 = (jnp.mean(ue * ue) + jnp.mean(ie * ie)
          + jnp.mean(ub * ub) + jnp.mean(ib * ib))
    loss_ref[0, 0] = jnp.mean(d * d) + _LAMBDA * l2


def _tc_small(ue, ie, ub, ib, y, gb2d):
    return pl.pallas_call(
        _small_body,
        in_specs=[
            pl.BlockSpec((_B, _HID), lambda: (0, 0)),
            pl.BlockSpec((_B, _HID), lambda: (0, 0)),
            pl.BlockSpec((_B,), lambda: (0,)),
            pl.BlockSpec((_B,), lambda: (0,)),
            pl.BlockSpec((_B,), lambda: (0,)),
            pl.BlockSpec((1, 1), lambda: (0, 0), memory_space=pltpu.SMEM),
        ],
        out_specs=[
            pl.BlockSpec((_B,), lambda: (0,)),
            pl.BlockSpec((1, 1), lambda: (0, 0), memory_space=pltpu.SMEM),
            pl.BlockSpec((_B,), lambda: (0,)),
        ],
        out_shape=[
            jax.ShapeDtypeStruct((_B,), jnp.float32),
            jax.ShapeDtypeStruct((1, 1), jnp.float32),
            jax.ShapeDtypeStruct((_B,), jnp.float32),
        ],
    )(ue, ie, ub, ib, y, gb2d)


def kernel(u, i, y, user_emb, item_emb, user_bias, item_bias, global_bias):
    et = user_emb.T
    eti = item_emb.T
    ue, ub = _sc_gather(et, u, user_bias, _NU)
    ie, ib = _sc_gather(eti, i, item_bias, _NI)
    gb2d = jnp.reshape(global_bias, (1, 1)).astype(jnp.float32)
    score_t = _tc_score(ue, eti, item_bias)
    s, loss, diff = _tc_small(ue, ie, ub, ib, y, gb2d)
    return s, score_t.T, jnp.reshape(loss, ()), diff


# speedup vs baseline: 1.0066x; 1.0032x over previous
import jax
import jax.numpy as jnp
from jax import lax
from jax.experimental import pallas as pl
from jax.experimental.pallas import tpu as pltpu
from jax.experimental.pallas import tpu_sc as plsc

_B = 1024
_HID = 32
_LAMBDA = 1e-05
_NU = 1000000
_NI = 100000

_NC = 2
_NS = 16
_NW = _NC * _NS
_BPW = _B // _NW

_RING = 16

_BN = 4096


def _make_gather_body(n_rows):
    cmax = n_rows // 128 - 1
    toff = (cmax + 1) * 128
    tw = n_rows - toff

    def body(et_hbm, idx_hbm, bias_hbm,
             e_out, b_out,
             idx_v, e_loc, tail, b_v, bsem, bufs, sems, tsem):
        wid = lax.axis_index("s") * _NC + lax.axis_index("c")
        base = wid * _BPW
        pltpu.sync_copy(idx_hbm.at[pl.ds(base, _BPW)], idx_v)
        cb = pltpu.async_copy(bias_hbm.at[idx_v], b_v, bsem)
        ctail = pltpu.async_copy(et_hbm.at[:, pl.ds(toff, tw)], tail, tsem)

        iota16 = lax.iota(jnp.int32, 16)

        def issue(j):
            chunk = idx_v[pl.ds((j // 16) * 16, 16)]
            xj = chunk[j % 16]
            c = jnp.minimum(xj // 128, cmax)
            off = pl.multiple_of(c * 128, 128)
            slot = j % _RING
            return pltpu.async_copy(et_hbm.at[:, pl.ds(off, 128)],
                                    bufs[slot], sems[slot])

        def extract(j):
            chunk = idx_v[pl.ds((j // 16) * 16, 16)]
            xj = chunk[j % 16]
            c = jnp.minimum(xj // 128, cmax)
            lane_m = jnp.minimum(xj - c * 128, 127)
            lane_t = jnp.clip(xj - toff, 0, tw - 1)
            in_tail = jnp.full((16,), xj >= toff, jnp.bool_)
            slot = j % _RING
            for h in range(2):
                rows = iota16 + h * 16
                vm = plsc.load_gather(
                    bufs[slot], [rows, jnp.full((16,), lane_m, jnp.int32)])
                vt = plsc.load_gather(
                    tail, [rows, jnp.full((16,), lane_t, jnp.int32)])
                e_loc[j, pl.ds(h * 16, 16)] = jnp.where(in_tail, vt, vm)

        pend = [None] * _RING
        for j in range(_RING):
            pend[j] = issue(j)
        ctail.wait()
        for j in range(_BPW):
            pend[j % _RING].wait()
            extract(j)
            nj = j + _RING
            if nj < _BPW:
                pend[nj % _RING] = issue(nj)

        pltpu.sync_copy(e_loc, e_out.at[pl.ds(base, _BPW), :])
        cb.wait()
        pltpu.sync_copy(b_v, b_out.at[pl.ds(base, _BPW)])

    return body, tw


def _sc_gather(et, idx, bias, n_rows):
    body, tw = _make_gather_body(n_rows)
    mesh = plsc.VectorSubcoreMesh(
        core_axis_name="c", subcore_axis_name="s",
        num_cores=_NC, num_subcores=_NS)
    f = pl.kernel(
        body,
        out_type=(
            jax.ShapeDtypeStruct((_B, _HID), jnp.float32),
            jax.ShapeDtypeStruct((_B,), jnp.float32),
        ),
        mesh=mesh,
        scratch_types=[
            pltpu.VMEM((_BPW,), jnp.int32),
            pltpu.VMEM((_BPW, _HID), jnp.float32),
            pltpu.VMEM((_HID, tw), jnp.float32),
            pltpu.VMEM((_BPW,), jnp.float32),
            pltpu.SemaphoreType.DMA,
            [pltpu.VMEM((_HID, 128), jnp.float32)] * _RING,
            [pltpu.SemaphoreType.DMA] * _RING,
            pltpu.SemaphoreType.DMA,
        ],
        compiler_params=pltpu.CompilerParams(use_tc_tiling_on_sc=True,
                                             needs_layout_passes=False),
    )
    return f(et, idx, bias)


def _item_gather_small_body(eti_hbm, i_hbm, ibias_hbm, ue_hbm, ub_hbm,
                            y_hbm, gb_hbm,
                            ie_out, ib_out, s_out, d_out, part_out,
                            idx_v, e_loc, et_loc, ue_loc, ut_loc, ub_loc,
                            y_loc, gb_loc, sd_loc, tail, b_v, bsem,
                            bufs, sems, tsem):
    cmax = _NI // 128 - 1
    toff = (cmax + 1) * 128
    tw = _NI - toff
    wid = lax.axis_index("s") * _NC + lax.axis_index("c")
    base = wid * _BPW
    pltpu.sync_copy(i_hbm.at[pl.ds(base, _BPW)], idx_v)
    cb = pltpu.async_copy(ibias_hbm.at[idx_v], b_v, bsem)
    ctail = pltpu.async_copy(eti_hbm.at[:, pl.ds(toff, tw)], tail, tsem)
    pltpu.sync_copy(ue_hbm.at[pl.ds(base, _BPW), :], ue_loc)
    pltpu.sync_copy(ub_hbm.at[pl.ds(base, _BPW)], ub_loc)
    pltpu.sync_copy(y_hbm.at[pl.ds(base, _BPW)], y_loc)
    pltpu.sync_copy(gb_hbm, gb_loc)

    iota16 = lax.iota(jnp.int32, 16)

    def issue(j):
        chunk = idx_v[pl.ds((j // 16) * 16, 16)]
        xj = chunk[j % 16]
        c = jnp.minimum(xj // 128, cmax)
        off = pl.multiple_of(c * 128, 128)
        slot = j % _RING
        return pltpu.async_copy(eti_hbm.at[:, pl.ds(off, 128)],
                                bufs[slot], sems[slot])

    def extract(j):
        chunk = idx_v[pl.ds((j // 16) * 16, 16)]
        xj = chunk[j % 16]
        c = jnp.minimum(xj // 128, cmax)
        lane_m = jnp.minimum(xj - c * 128, 127)
        lane_t = jnp.clip(xj - toff, 0, tw - 1)
        in_tail = jnp.full((16,), xj >= toff, jnp.bool_)
        slot = j % _RING
        for h in range(2):
            rows = iota16 + h * 16
            vm = plsc.load_gather(
                bufs[slot], [rows, jnp.full((16,), lane_m, jnp.int32)])
            vt = plsc.load_gather(
                tail, [rows, jnp.full((16,), lane_t, jnp.int32)])
            val = jnp.where(in_tail, vt, vm)
            e_loc[j, pl.ds(h * 16, 16)] = val
            plsc.store_scatter(et_loc, [rows, jnp.full((16,), j, jnp.int32)],
                               val)

    pend = [None] * _RING
    for j in range(_RING):
        pend[j] = issue(j)
    ctail.wait()
    for j in range(_BPW):
        pend[j % _RING].wait()
        extract(j)
        nj = j + _RING
        if nj < _BPW:
            pend[nj % _RING] = issue(nj)

    pltpu.sync_copy(e_loc, ie_out.at[pl.ds(base, _BPW), :])
    cb.wait()
    pltpu.sync_copy(b_v, ib_out.at[pl.ds(base, _BPW)])

    for r in range(_BPW):
        for h in range(2):
            v = ue_loc[r, pl.ds(h * 16, 16)]
            plsc.store_scatter(ut_loc,
                               [iota16 + h * 16,
                                jnp.full((16,), r, jnp.int32)], v)

    zeros = jnp.zeros((16,), jnp.float32)
    acc = [zeros, zeros]
    pue = zeros
    pie = zeros
    for f in range(_HID):
        for g in range(2):
            vu = ut_loc[f, pl.ds(g * 16, 16)]
            vi = et_loc[f, pl.ds(g * 16, 16)]
            acc[g] = acc[g] + vu * vi
            pue = pue + vu * vu
            pie = pie + vi * vi
    pd2 = zeros
    pub2 = zeros
    pib2 = zeros
    gb = gb_loc[...]
    for g in range(2):
        ubg = ub_loc[pl.ds(g * 16, 16)]
        ibg = b_v[pl.ds(g * 16, 16)]
        sg = acc[g] + ubg + ibg + gb
        dg = sg - y_loc[pl.ds(g * 16, 16)]
        sd_loc[pl.ds(g * 16, 16)] = sg
        sd_loc[pl.ds(_BPW + g * 16, 16)] = dg
        pd2 = pd2 + dg * dg
        pub2 = pub2 + ubg * ubg
        pib2 = pib2 + ibg * ibg
    pltpu.sync_copy(sd_loc.at[pl.ds(0, _BPW)], s_out.at[pl.ds(base, _BPW)])
    pltpu.sync_copy(sd_loc.at[pl.ds(_BPW, _BPW)],
                    d_out.at[pl.ds(base, _BPW)])
    parts = [pd2, pue, pie, pub2, pib2]
    for k in range(5):
        sd_loc[pl.ds(2 * _BPW + k * 16, 16)] = parts[k]
    pltpu.sync_copy(sd_loc.at[pl.ds(2 * _BPW, 80)],
                    part_out.at[pl.ds(wid * 80, 80)])


def _sc_item_gather_small(eti, i, item_bias, ue, ub, y, gb16):
    mesh = plsc.VectorSubcoreMesh(
        core_axis_name="c", subcore_axis_name="s",
        num_cores=_NC, num_subcores=_NS)
    tw = _NI - (_NI // 128) * 128
    f = pl.kernel(
        _item_gather_small_body,
        out_type=(
            jax.ShapeDtypeStruct((_B, _HID), jnp.float32),
            jax.ShapeDtypeStruct((_B,), jnp.float32),
            jax.ShapeDtypeStruct((_B,), jnp.float32),
            jax.ShapeDtypeStruct((_B,), jnp.float32),
            jax.ShapeDtypeStruct((_NW * 80,), jnp.float32),
        ),
        mesh=mesh,
        scratch_types=[
            pltpu.VMEM((_BPW,), jnp.int32),
            pltpu.VMEM((_BPW, _HID), jnp.float32),
            pltpu.VMEM((_HID, _BPW), jnp.float32),
            pltpu.VMEM((_BPW, _HID), jnp.float32),
            pltpu.VMEM((_HID, _BPW), jnp.float32),
            pltpu.VMEM((_BPW,), jnp.float32),
            pltpu.VMEM((_BPW,), jnp.float32),
            pltpu.VMEM((16,), jnp.float32),
            pltpu.VMEM((2 * _BPW + 80,), jnp.float32),
            pltpu.VMEM((_HID, tw), jnp.float32),
            pltpu.VMEM((_BPW,), jnp.float32),
            pltpu.SemaphoreType.DMA,
            [pltpu.VMEM((_HID, 128), jnp.float32)] * _RING,
            [pltpu.SemaphoreType.DMA] * _RING,
            pltpu.SemaphoreType.DMA,
        ],
        compiler_params=pltpu.CompilerParams(use_tc_tiling_on_sc=True,
                                             needs_layout_passes=False),
    )
    return f(eti, i, item_bias, ue, ub, y, gb16)


def _loss_body(p_ref, loss_ref):
    p = p_ref[...]
    tot = []
    for k in range(5):
        acc = jnp.zeros((16,), jnp.float32)
        for w in range(_NW):
            a = w * 80 + k * 16
            acc = acc + p[a:a + 16]
        tot.append(jnp.sum(acc))
    loss_ref[0, 0] = (tot[0] / _B
                      + _LAMBDA * (tot[1] / (_B * _HID)
                                   + tot[2] / (_B * _HID)
                                   + tot[3] / _B + tot[4] / _B))


def _tc_loss(part):
    return pl.pallas_call(
        _loss_body,
        in_specs=[pl.BlockSpec((_NW * 80,), lambda: (0,))],
        out_specs=pl.BlockSpec((1, 1), lambda: (0, 0),
                               memory_space=pltpu.SMEM),
        out_shape=jax.ShapeDtypeStruct((1, 1), jnp.float32),
    )(part)


def _score_body(ue_ref, eti_ref, ibias_ref, score_ref, at_ref):
    pid = pl.program_id(0)

    @pl.when(pid == 0)
    def _prep():
        at_ref[...] = ue_ref[...].T

    sc = lax.dot_general(eti_ref[...], at_ref[...],
                         (((0,), (0,)), ((), ())),
                         preferred_element_type=jnp.float32)
    score_ref[...] = sc + ibias_ref[...][:, None]


def _tc_score(ue, eti, item_bias):
    grid = (pl.cdiv(_NI, _BN),)
    return pl.pallas_call(
        _score_body,
        grid=grid,
        in_specs=[
            pl.BlockSpec((_B, _HID), lambda n: (0, 0)),
            pl.BlockSpec((_HID, _BN), lambda n: (0, n)),
            pl.BlockSpec((_BN,), lambda n: (n,)),
        ],
        out_specs=pl.BlockSpec((_BN, _B), lambda n: (n, 0)),
        out_shape=jax.ShapeDtypeStruct((_NI, _B), jnp.float32),
        scratch_shapes=[pltpu.VMEM((_HID, _B), jnp.float32)],
        compiler_params=pltpu.CompilerParams(
            dimension_semantics=("parallel",),
        ),
    )(ue, eti, item_bias)


def kernel(u, i, y, user_emb, item_emb, user_bias, item_bias, global_bias):
    et = user_emb.T
    eti = item_emb.T
    gb16 = jnp.broadcast_to(jnp.reshape(global_bias, (1,)),
                            (16,)).astype(jnp.float32)
    ue, ub = _sc_gather(et, u, user_bias, _NU)
    ie, ib, s, diff, part = _sc_item_gather_small(eti, i, item_bias,
                                                  ue, ub, y, gb16)
    score_t = _tc_score(ue, eti, item_bias)
    loss = _tc_loss(part)
    return s, score_t.T, jnp.reshape(loss, ()), diff
